# baseline (speedup 1.0000x reference)
import functools
import math

import jax
import jax.numpy as jnp
from jax import lax
from jax.experimental import pallas as pl
from jax.experimental.pallas import tpu as pltpu
from jax.experimental.pallas import tpu_sc as plsc

_NC = 2
_NS = 16
_NW = _NC * _NS
_LANES = 16
_G = 128


def _make_emb_kernel(N, V, D):
    per_w = N // _NW
    ngrp = per_w // _G
    scale = math.sqrt(D)
    mesh = plsc.VectorSubcoreMesh(core_axis_name="c", subcore_axis_name="s")

    @functools.partial(
        pl.kernel,
        out_type=jax.ShapeDtypeStruct((N, D), jnp.float32),
        mesh=mesh,
        scratch_types=[
            pltpu.VMEM((_G,), jnp.int32),
            pltpu.VMEM((_G, D), jnp.float32),
            pltpu.SemaphoreType.DMA,
        ],
        compiler_params=pltpu.CompilerParams(use_tc_tiling_on_sc=False),
    )
    def emb(table_hbm, idx_hbm, out_hbm, idx_v, rows_v, sem):
        wid = lax.axis_index("s") * _NC + lax.axis_index("c")
        base = wid * per_w

        def do_group(g, carry):
            row0 = base + g * _G
            pltpu.sync_copy(idx_hbm.at[pl.ds(row0, _G)], idx_v)
            pltpu.async_copy(table_hbm.at[idx_v], rows_v, sem).wait()

            def scale_row(i, c):
                for j in range(D // _LANES):
                    sl = pl.ds(j * _LANES, _LANES)
                    rows_v[i, sl] = rows_v[i, sl] * scale
                return c

            lax.fori_loop(0, _G, scale_row, 0)
            pltpu.sync_copy(rows_v, out_hbm.at[pl.ds(row0, _G)])
            return carry

        lax.fori_loop(0, ngrp, do_group, 0)

    return emb


def kernel(input, table):
    B, L = input.shape
    V, D = table.shape
    N = B * L
    idx_flat = input.reshape(N).astype(jnp.int32)
    emb = _make_emb_kernel(N, V, D)
    out = emb(table, idx_flat)
    return out.reshape(B, L, D)


# speedup vs baseline: 1.1070x; 1.1070x over previous
import functools
import math

import jax
import jax.numpy as jnp
from jax import lax
from jax.experimental import pallas as pl
from jax.experimental.pallas import tpu as pltpu
from jax.experimental.pallas import tpu_sc as plsc

_NC = 2
_NS = 16
_NW = _NC * _NS
_LANES = 16
_G = 128
_NBUF = 5


def _make_emb_kernel(N, V, D):
    per_w = N // _NW
    ngrp = per_w // _G
    nouter = ngrp // _NBUF
    scale = math.sqrt(D)
    mesh = plsc.VectorSubcoreMesh(core_axis_name="c", subcore_axis_name="s")

    scratch = (
        [pltpu.VMEM((ngrp, _G), jnp.int32)]
        + [pltpu.VMEM((_G, D), jnp.float32) for _ in range(2 * _NBUF)]
        + [pltpu.SemaphoreType.DMA for _ in range(2 * _NBUF)]
    )

    @functools.partial(
        pl.kernel,
        out_type=jax.ShapeDtypeStruct((N, D), jnp.float32),
        mesh=mesh,
        scratch_types=scratch,
        compiler_params=pltpu.CompilerParams(use_tc_tiling_on_sc=False),
    )
    def emb(table_hbm, idx_hbm, out_hbm, idx_v, *bufs):
        rows_in = bufs[:_NBUF]
        rows_out = bufs[_NBUF:2 * _NBUF]
        sem_g = bufs[2 * _NBUF:3 * _NBUF]
        sem_o = bufs[3 * _NBUF:]
        wid = lax.axis_index("s") * _NC + lax.axis_index("c")
        base = wid * per_w

        pltpu.sync_copy(idx_hbm.at[wid], idx_v)

        def start_gather(b, g):
            pltpu.async_copy(table_hbm.at[idx_v.at[g]], rows_in[b], sem_g[b])

        def wait_gather(b, g):
            pltpu.make_async_copy(
                table_hbm.at[idx_v.at[g]], rows_in[b], sem_g[b]).wait()

        def start_out(b, g):
            pltpu.async_copy(
                rows_out[b], out_hbm.at[pl.ds(base + g * _G, _G)], sem_o[b])

        def wait_out(b, g):
            pltpu.make_async_copy(
                rows_out[b], out_hbm.at[pl.ds(base + g * _G, _G)], sem_o[b]).wait()

        for b in range(_NBUF):
            start_gather(b, b)

        def outer(t, carry):
            for b in range(_NBUF):
                g = t * _NBUF + b
                wait_gather(b, g)

                @pl.when(t > 0)
                def _():
                    wait_out(b, g - _NBUF)

                def scale_row(i, c):
                    for j in range(D // _LANES):
                        sl = pl.ds(j * _LANES, _LANES)
                        rows_out[b][i, sl] = rows_in[b][i, sl] * scale
                    return c

                lax.fori_loop(0, _G, scale_row, 0)
                start_out(b, g)

                @pl.when(t < nouter - 1)
                def _():
                    start_gather(b, g + _NBUF)
            return carry

        lax.fori_loop(0, nouter, outer, 0)

        for b in range(_NBUF):
            wait_out(b, (nouter - 1) * _NBUF + b)

    return emb


def kernel(input, table):
    B, L = input.shape
    V, D = table.shape
    N = B * L
    per_w = N // _NW
    ngrp = per_w // _G
    idx = input.reshape(_NW, ngrp, _G).astype(jnp.int32)
    emb = _make_emb_kernel(N, V, D)
    out = emb(table, idx)
    return out.reshape(B, L, D)


# speedup vs baseline: 1.5856x; 1.4324x over previous
import functools
import math

import jax
import jax.numpy as jnp
from jax import lax
from jax.experimental import pallas as pl
from jax.experimental.pallas import tpu as pltpu
from jax.experimental.pallas import tpu_sc as plsc

_NC = 2
_NS = 16
_NW = _NC * _NS
_LANES = 16
_C = 400


def _make_emb_kernel(N, V, D):
    per_w = N // _NW
    nchunk = per_w // _C
    mesh = plsc.VectorSubcoreMesh(core_axis_name="c", subcore_axis_name="s")

    scratch = (
        [pltpu.VMEM((_C,), jnp.int32) for _ in range(2)]
        + [pltpu.VMEM((_C, D), jnp.float32) for _ in range(2)]
        + [pltpu.SemaphoreType.DMA for _ in range(4)]
    )

    @functools.partial(
        pl.kernel,
        out_type=jax.ShapeDtypeStruct((N, D), jnp.float32),
        mesh=mesh,
        scratch_types=scratch,
    )
    def emb(table_hbm, ids_hbm, out_hbm, idx0, idx1, rows0, rows1,
            g0, g1, o0, o1):
        idx = (idx0, idx1)
        rows = (rows0, rows1)
        sem_g = (g0, g1)
        sem_o = (o0, o1)
        wid = lax.axis_index("s") * _NC + lax.axis_index("c")
        base = wid * per_w

        def fire_chunk(c, buf):
            pltpu.sync_copy(ids_hbm.at[pl.ds(base + c * _C, _C)], idx[buf])

            def fire16(k, carry):
                vec = idx[buf][pl.ds(k * _LANES, _LANES)]
                for lane in range(_LANES):
                    r = vec[lane]
                    pltpu.async_copy(
                        table_hbm.at[pl.ds(r, 1)],
                        rows[buf].at[pl.ds(k * _LANES + lane, 1)],
                        sem_g[buf],
                    )
                return carry

            lax.fori_loop(0, _C // _LANES, fire16, 0)

        def drain_chunk(buf):
            pltpu.make_async_copy(
                table_hbm.at[pl.ds(0, _C)], rows[buf], sem_g[buf]).wait()

        def fire_out(c, buf):
            pltpu.async_copy(
                rows[buf], out_hbm.at[pl.ds(base + c * _C, _C)], sem_o[buf])

        def wait_out(c, buf):
            pltpu.make_async_copy(
                rows[buf], out_hbm.at[pl.ds(base + c * _C, _C)],
                sem_o[buf]).wait()

        fire_chunk(0, 0)

        def pair_body(t, carry):
            for buf in (1, 0):
                c = 2 * t + (1 if buf == 1 else 2)

                @pl.when(c >= 2)
                def _():
                    wait_out(c - 2, buf)

                @pl.when(c < nchunk)
                def _():
                    fire_chunk(c, buf)
                drain_chunk(1 - buf)
                fire_out(c - 1, 1 - buf)
            return carry

        lax.fori_loop(0, nchunk // 2, pair_body, 0)
        wait_out(nchunk - 1, (nchunk - 1) % 2)

    return emb


def kernel(input, table):
    B, L = input.shape
    V, D = table.shape
    N = B * L
    emb = _make_emb_kernel(N, V, D)
    out = emb(table, input.reshape(N).astype(jnp.int32))
    return out.reshape(B, L, D) * math.sqrt(D)
